# Initial kernel scaffold; baseline (speedup 1.0000x reference)
import jax
import jax.numpy as jnp
from jax.experimental import pallas as pl


def kernel(x, edge_index, W1, b1, W2, b2):
    raise NotImplementedError("write your pallas kernel here")


# speedup vs baseline: 348.0349x; 348.0349x over previous
import functools

import jax
import jax.numpy as jnp
from jax import lax
from jax.experimental import pallas as pl
from jax.experimental.pallas import tpu as pltpu
from jax.experimental.pallas import tpu_sc as plsc

NC = 2
NS = 16
L = 16
NW = NC * NS


def _largest_divisor(n, cap, mult):
    d = 0
    for c in range(mult, cap + 1, mult):
        if n % c == 0:
            d = c
    return d


def _hist_body(np_, ew, ch, stripe, dst_hbm, out_hbm, idxv, onesv, tmpv, deg_sp):
    c = lax.axis_index("c")
    s = lax.axis_index("s")
    wid = s * NC + c

    def fill_ones(j, _):
        onesv[pl.ds(j * L, L)] = jnp.full((L,), 1.0, jnp.float32)
        return 0

    lax.fori_loop(0, ch // L, fill_ones, 0)

    def fill_zero(j, _):
        tmpv[pl.ds(j * L, L)] = jnp.zeros((L,), jnp.float32)
        return 0

    lax.fori_loop(0, stripe // L, fill_zero, 0)
    pltpu.sync_copy(tmpv, deg_sp.at[pl.ds(s * stripe, stripe)])
    plsc.subcore_barrier()

    def chunk(i, _):
        base = wid * ew + i * ch
        pltpu.sync_copy(dst_hbm.at[pl.ds(base, ch)], idxv)
        pltpu.sync_copy(onesv, deg_sp.at[idxv], add=True)
        return 0

    lax.fori_loop(0, ew // ch, chunk, 0)
    plsc.subcore_barrier()
    pltpu.sync_copy(deg_sp.at[pl.ds(s * stripe, stripe)], tmpv)
    pltpu.sync_copy(tmpv, out_hbm.at[pl.ds(c * np_ + s * stripe, stripe)])


def _make_hist(np_, e):
    ew = e // NW
    ch = _largest_divisor(ew, 20000, 16)
    stripe = np_ // NS
    mesh = plsc.VectorSubcoreMesh(core_axis_name="c", subcore_axis_name="s")
    params = pltpu.CompilerParams(needs_layout_passes=False)
    return pl.kernel(
        functools.partial(_hist_body, np_, ew, ch, stripe),
        out_type=jax.ShapeDtypeStruct((NC * np_,), jnp.float32),
        mesh=mesh,
        compiler_params=params,
        scratch_types=[
            pltpu.VMEM((ch,), jnp.int32),
            pltpu.VMEM((ch,), jnp.float32),
            pltpu.VMEM((stripe,), jnp.float32),
            pltpu.VMEM_SHARED((np_,), jnp.float32),
        ],
    )


def _pass_body(np_, ew, cp, stripe, src_hbm, dst_hbm, tab_hbm, out_hbm,
               tabv, srcv, dstv, valsv, agg_sp):
    c = lax.axis_index("c")
    s = lax.axis_index("s")
    wid = s * NC + c

    pltpu.sync_copy(tab_hbm, tabv)

    def fill_zero(j, _):
        valsv[pl.ds(j * L, L)] = jnp.zeros((L,), jnp.float32)
        return 0

    lax.fori_loop(0, stripe // L, fill_zero, 0)
    pltpu.sync_copy(valsv.at[pl.ds(0, stripe)], agg_sp.at[pl.ds(s * stripe, stripe)])
    plsc.subcore_barrier()

    def chunk(i, _):
        base = wid * ew + i * cp
        pltpu.sync_copy(src_hbm.at[pl.ds(base, cp)], srcv)
        pltpu.sync_copy(dst_hbm.at[pl.ds(base, cp)], dstv)

        def gather(j, _):
            idx = srcv[pl.ds(j * L, L)]
            valsv[pl.ds(j * L, L)] = plsc.load_gather(tabv, [idx])
            return 0

        lax.fori_loop(0, cp // L, gather, 0)
        pltpu.sync_copy(valsv, agg_sp.at[dstv], add=True)
        return 0

    lax.fori_loop(0, ew // cp, chunk, 0)
    plsc.subcore_barrier()
    pltpu.sync_copy(agg_sp.at[pl.ds(s * stripe, stripe)], valsv.at[pl.ds(0, stripe)])
    pltpu.sync_copy(valsv.at[pl.ds(0, stripe)],
                    out_hbm.at[pl.ds(c * np_ + s * stripe, stripe)])


def _make_pass(np_, e):
    ew = e // NW
    cap = min(10000, (2090000 - 17 * np_) // (3 * NS))
    cp = _largest_divisor(ew, cap, 16)
    stripe = np_ // NS
    mesh = plsc.VectorSubcoreMesh(core_axis_name="c", subcore_axis_name="s")
    params = pltpu.CompilerParams(needs_layout_passes=False)
    return pl.kernel(
        functools.partial(_pass_body, np_, ew, cp, stripe),
        out_type=jax.ShapeDtypeStruct((NC * np_,), jnp.float32),
        mesh=mesh,
        compiler_params=params,
        scratch_types=[
            pltpu.VMEM((np_,), jnp.float32),
            pltpu.VMEM((cp,), jnp.int32),
            pltpu.VMEM((cp,), jnp.int32),
            pltpu.VMEM((cp,), jnp.float32),
            pltpu.VMEM_SHARED((np_,), jnp.float32),
        ],
    )


def _ew1_body(degp_ref, x_ref, u_ref, dinv_ref):
    deg = degp_ref[0] + degp_ref[1] + 1.0
    di = lax.rsqrt(deg)
    dinv_ref[...] = di
    u_ref[...] = x_ref[...] * di


def _ew2_body(h, abc_ref, aggp_ref, u_ref, dinv_ref, v_ref):
    di = dinv_ref[...]
    s1 = di * (aggp_ref[0] + aggp_ref[1] + u_ref[...])
    acc = jnp.zeros_like(s1)
    for k in range(h):
        acc = acc + abc_ref[2, k] * jnp.maximum(s1 * abc_ref[0, k] + abc_ref[1, k], 0.0)
    v_ref[...] = acc * di


def _ew3_body(b2_ref, aggp_ref, v_ref, dinv_ref, out_ref):
    t = dinv_ref[...] * (aggp_ref[0] + aggp_ref[1] + v_ref[...]) + b2_ref[0]
    out_ref[...] = jnp.clip(t, -0.5, 9.5)


def kernel(x, edge_index, W1, b1, W2, b2):
    n = x.shape[0]
    e = edge_index.shape[1]
    h = W1.shape[1]
    np_ = ((n + 127) // 128) * 128
    rows = np_ // 128

    src = edge_index[0]
    dst = edge_index[1]
    xp = jnp.pad(x[:, 0], (0, np_ - n)).reshape(rows, 128)

    hist = _make_hist(np_, e)
    edge_pass = _make_pass(np_, e)

    degp = hist(dst)

    u, dinv = pl.pallas_call(
        _ew1_body,
        in_specs=[pl.BlockSpec(memory_space=pltpu.VMEM)] * 2,
        out_specs=[pl.BlockSpec(memory_space=pltpu.VMEM)] * 2,
        out_shape=[jax.ShapeDtypeStruct((rows, 128), jnp.float32)] * 2,
    )(degp.reshape(NC, rows, 128), xp)

    agg1p = edge_pass(src, dst, u.reshape(np_))

    abc = jnp.stack([W1[0, :], b1, W2[:, 0]])
    v = pl.pallas_call(
        functools.partial(_ew2_body, h),
        in_specs=[pl.BlockSpec(memory_space=pltpu.SMEM)]
        + [pl.BlockSpec(memory_space=pltpu.VMEM)] * 3,
        out_specs=pl.BlockSpec(memory_space=pltpu.VMEM),
        out_shape=jax.ShapeDtypeStruct((rows, 128), jnp.float32),
    )(abc, agg1p.reshape(NC, rows, 128), u, dinv)

    agg2p = edge_pass(src, dst, v.reshape(np_))

    out = pl.pallas_call(
        _ew3_body,
        in_specs=[pl.BlockSpec(memory_space=pltpu.SMEM)]
        + [pl.BlockSpec(memory_space=pltpu.VMEM)] * 3,
        out_specs=pl.BlockSpec(memory_space=pltpu.VMEM),
        out_shape=jax.ShapeDtypeStruct((rows, 128), jnp.float32),
    )(b2, agg2p.reshape(NC, rows, 128), v, dinv)

    return out.reshape(np_)[:n, None]
